# baseline (speedup 1.0000x reference)
import functools

import jax
import jax.numpy as jnp
from jax import lax
from jax.experimental import pallas as pl
from jax.experimental.pallas import tpu as pltpu
from jax.experimental.pallas import tpu_sc as plsc

N_NODES = 10000
N_EDGES = 320000
NODE_DIM = 128
EDIM = 16

SC_CORES = 2
SC_SUBCORES = 16
NW = SC_CORES * SC_SUBCORES
EDGES_PER_W = N_EDGES // NW
GCHUNK = 2000

ROW_BLK = 1000
EDGE_BLK = 6400


def _proj_body(s_ref, wa_ref, wb_ref, pa_ref, pb_ref):
    s = s_ref[...]
    pa_ref[...] = jnp.dot(s, wa_ref[...], preferred_element_type=jnp.float32)
    pb_ref[...] = jnp.dot(s, wb_ref[...], preferred_element_type=jnp.float32)


def _project_nodes(scalars, w1a, w1b):
    grid = N_NODES // ROW_BLK
    return pl.pallas_call(
        _proj_body,
        grid=(grid,),
        in_specs=[
            pl.BlockSpec((ROW_BLK, NODE_DIM), lambda i: (i, 0)),
            pl.BlockSpec((NODE_DIM, EDIM), lambda i: (0, 0)),
            pl.BlockSpec((NODE_DIM, EDIM), lambda i: (0, 0)),
        ],
        out_specs=[
            pl.BlockSpec((ROW_BLK, EDIM), lambda i: (i, 0)),
            pl.BlockSpec((ROW_BLK, EDIM), lambda i: (i, 0)),
        ],
        out_shape=[
            jax.ShapeDtypeStruct((N_NODES, EDIM), jnp.float32),
            jax.ShapeDtypeStruct((N_NODES, EDIM), jnp.float32),
        ],
    )(scalars, w1a, w1b)


def _sc_gather_body(ps_hbm, pr_hbm, snd_hbm, rcv_hbm, gt_hbm,
                    idx_s, idx_r, rows_s, rows_r, gt_v, sem_s, sem_r):
    wid = lax.axis_index("s") * SC_CORES + lax.axis_index("c")
    base = wid * EDGES_PER_W
    lanes = lax.iota(jnp.int32, 16)

    def chunk(i, carry):
        off = base + i * GCHUNK
        pltpu.sync_copy(snd_hbm.at[pl.ds(off, GCHUNK)], idx_s)
        pltpu.sync_copy(rcv_hbm.at[pl.ds(off, GCHUNK)], idx_r)
        cs = pltpu.async_copy(ps_hbm.at[idx_s], rows_s, sem_s)
        cr = pltpu.async_copy(pr_hbm.at[idx_r], rows_r, sem_r)
        cs.wait()
        cr.wait()

        @plsc.parallel_loop(0, GCHUNK // 16, unroll=8)
        def blk(b):
            row_idx = b * 16 + lanes
            for j in range(EDIM):
                col_idx = jnp.full((16,), j, dtype=jnp.int32)
                v = (plsc.load_gather(rows_s, [row_idx, col_idx])
                     + plsc.load_gather(rows_r, [row_idx, col_idx]))
                gt_v[j, pl.ds(b * 16, 16)] = v
        pltpu.sync_copy(gt_v, gt_hbm.at[:, pl.ds(off, GCHUNK)])
        return carry

    lax.fori_loop(0, EDGES_PER_W // GCHUNK, chunk, 0)


def _sc_gather(p_send, p_recv, sender, receiver):
    mesh = plsc.VectorSubcoreMesh(
        core_axis_name="c", subcore_axis_name="s",
        num_cores=SC_CORES, num_subcores=SC_SUBCORES,
    )
    f = pl.kernel(
        _sc_gather_body,
        out_type=jax.ShapeDtypeStruct((EDIM, N_EDGES), jnp.float32),
        mesh=mesh,
        scratch_types=[
            pltpu.VMEM((GCHUNK,), jnp.int32),
            pltpu.VMEM((GCHUNK,), jnp.int32),
            pltpu.VMEM((GCHUNK, EDIM), jnp.float32),
            pltpu.VMEM((GCHUNK, EDIM), jnp.float32),
            pltpu.VMEM((EDIM, GCHUNK), jnp.float32),
            pltpu.SemaphoreType.DMA,
            pltpu.SemaphoreType.DMA,
        ],
        compiler_params=pltpu.CompilerParams(use_tc_tiling_on_sc=False,
                                             needs_layout_passes=False),
    )
    return f(p_send, p_recv, sender, receiver)


def _epilogue_body(gt_ref, est_ref, el_ref, w1ct_ref, wlt_ref, b1t_ref,
                   w2t_ref, b2t_ref, out_ref):
    el_row = el_ref[pl.ds(pl.program_id(0), 1), :]
    z = (gt_ref[...]
         + jnp.dot(w1ct_ref[...], est_ref[...], preferred_element_type=jnp.float32)
         + wlt_ref[...] * el_row
         + b1t_ref[...])
    h = z * jax.nn.sigmoid(z)
    out_ref[...] = jnp.dot(w2t_ref[...], h,
                           preferred_element_type=jnp.float32) + b2t_ref[...]


def _epilogue(gt, es_t, el2, w1c_t, wl_t, b1_t, w2_t, b2_t):
    grid = N_EDGES // EDGE_BLK
    cblk = lambda i: (0, i)
    zblk = lambda i: (0, 0)
    return pl.pallas_call(
        _epilogue_body,
        grid=(grid,),
        in_specs=[
            pl.BlockSpec((EDIM, EDGE_BLK), cblk),
            pl.BlockSpec((EDIM, EDGE_BLK), cblk),
            pl.BlockSpec((N_EDGES // EDGE_BLK, EDGE_BLK), lambda i: (0, 0)),
            pl.BlockSpec((EDIM, EDIM), zblk),
            pl.BlockSpec((EDIM, 1), zblk),
            pl.BlockSpec((EDIM, 1), zblk),
            pl.BlockSpec((EDIM, EDIM), zblk),
            pl.BlockSpec((EDIM, 1), zblk),
        ],
        out_specs=pl.BlockSpec((EDIM, EDGE_BLK), cblk),
        out_shape=jax.ShapeDtypeStruct((EDIM, N_EDGES), jnp.float32),
    )(gt, es_t, el2, w1c_t, wl_t, b1_t, w2_t, b2_t)


@jax.jit
def kernel(scalars, edge_index, edge_len, edge_state, W1, b1, W2, b2):
    sender = edge_index[0].astype(jnp.int32)
    receiver = edge_index[1].astype(jnp.int32)
    w1a = W1[:NODE_DIM]
    w1b = W1[NODE_DIM:2 * NODE_DIM]
    w1c_t = W1[2 * NODE_DIM:2 * NODE_DIM + EDIM].T
    wl_t = W1[2 * NODE_DIM + EDIM:].T
    b1_t = b1.reshape(EDIM, 1)
    b2_t = b2.reshape(EDIM, 1)

    p_send, p_recv = _project_nodes(scalars, w1a, w1b)
    gt = _sc_gather(p_send, p_recv, sender, receiver)
    es_t = edge_state.T
    el2 = edge_len.reshape(N_EDGES // EDGE_BLK, EDGE_BLK)
    out_t = _epilogue(gt, es_t, el2, w1c_t, wl_t, b1_t, W2.T, b2_t)
    return out_t.T


# speedup vs baseline: 1.1300x; 1.1300x over previous
import functools

import jax
import jax.numpy as jnp
from jax import lax
from jax.experimental import pallas as pl
from jax.experimental.pallas import tpu as pltpu
from jax.experimental.pallas import tpu_sc as plsc

N_NODES = 10000
N_EDGES = 320000
NODE_DIM = 128
EDIM = 16

SC_CORES = 2
SC_SUBCORES = 16
NW = SC_CORES * SC_SUBCORES
EDGES_PER_W = N_EDGES // NW
GCHUNK = 1000
NCHUNK = EDGES_PER_W // GCHUNK

ROW_BLK = 1000
EDGE_BLK = 6400


def _proj_body(s_ref, wa_ref, wb_ref, pa_ref, pb_ref):
    s = s_ref[...]
    pa_ref[...] = jnp.dot(s, wa_ref[...], preferred_element_type=jnp.float32)
    pb_ref[...] = jnp.dot(s, wb_ref[...], preferred_element_type=jnp.float32)


def _project_nodes(scalars, w1a, w1b):
    grid = N_NODES // ROW_BLK
    return pl.pallas_call(
        _proj_body,
        grid=(grid,),
        in_specs=[
            pl.BlockSpec((ROW_BLK, NODE_DIM), lambda i: (i, 0)),
            pl.BlockSpec((NODE_DIM, EDIM), lambda i: (0, 0)),
            pl.BlockSpec((NODE_DIM, EDIM), lambda i: (0, 0)),
        ],
        out_specs=[
            pl.BlockSpec((ROW_BLK, EDIM), lambda i: (i, 0)),
            pl.BlockSpec((ROW_BLK, EDIM), lambda i: (i, 0)),
        ],
        out_shape=[
            jax.ShapeDtypeStruct((N_NODES, EDIM), jnp.float32),
            jax.ShapeDtypeStruct((N_NODES, EDIM), jnp.float32),
        ],
    )(scalars, w1a, w1b)


def _sc_gather_body(ps_hbm, pr_hbm, snd_hbm, rcv_hbm, gt_hbm,
                    idx_s0, idx_r0, rows_s0, rows_r0, gt_v0,
                    idx_s1, idx_r1, rows_s1, rows_r1, gt_v1,
                    sem_s0, sem_r0, sem_s1, sem_r1, sem_w0, sem_w1):
    wid = lax.axis_index("s") * SC_CORES + lax.axis_index("c")
    base = wid * EDGES_PER_W
    lanes = lax.iota(jnp.int32, 16)
    bufs = ((idx_s0, idx_r0, rows_s0, rows_r0, gt_v0, sem_s0, sem_r0, sem_w0),
            (idx_s1, idx_r1, rows_s1, rows_r1, gt_v1, sem_s1, sem_r1, sem_w1))

    def stage(i, slot):
        idx_s, idx_r, rows_s, rows_r, _, sem_s, sem_r, _ = bufs[slot]
        off = base + i * GCHUNK
        pltpu.sync_copy(snd_hbm.at[pl.ds(off, GCHUNK)], idx_s)
        pltpu.sync_copy(rcv_hbm.at[pl.ds(off, GCHUNK)], idx_r)
        pltpu.async_copy(ps_hbm.at[idx_s], rows_s, sem_s)
        pltpu.async_copy(pr_hbm.at[idx_r], rows_r, sem_r)

    def drainwb(slot):
        _, _, _, _, gt_v, _, _, sem_w = bufs[slot]
        pltpu.make_async_copy(gt_hbm.at[:, pl.ds(0, GCHUNK)], gt_v, sem_w).wait()

    def work(i, slot):
        idx_s, idx_r, rows_s, rows_r, gt_v, sem_s, sem_r, sem_w = bufs[slot]
        off = base + i * GCHUNK
        pltpu.make_async_copy(ps_hbm.at[idx_s], rows_s, sem_s).wait()
        pltpu.make_async_copy(pr_hbm.at[idx_r], rows_r, sem_r).wait()

        @plsc.parallel_loop(0, GCHUNK // 16, unroll=4)
        def blk(b):
            row_idx = b * 16 + lanes
            for j in range(EDIM):
                col_idx = jnp.full((16,), j, dtype=jnp.int32)
                v = (plsc.load_gather(rows_s, [row_idx, col_idx])
                     + plsc.load_gather(rows_r, [row_idx, col_idx]))
                gt_v[j, pl.ds(b * 16, 16)] = v
        pltpu.async_copy(gt_v, gt_hbm.at[:, pl.ds(off, GCHUNK)], sem_w)

    stage(0, 0)

    def step(pair, carry):
        for b in range(2):
            i = pair * 2 + b
            slot = b
            nxt = 1 - b

            @pl.when(i + 1 < NCHUNK)
            def _():
                @pl.when(i >= 1)
                def _():
                    drainwb(nxt)
                stage(i + 1, nxt)

            work(i, slot)
        return carry

    lax.fori_loop(0, NCHUNK // 2, step, 0)
    drainwb(0)
    drainwb(1)


def _sc_gather(p_send, p_recv, sender, receiver):
    mesh = plsc.VectorSubcoreMesh(
        core_axis_name="c", subcore_axis_name="s",
        num_cores=SC_CORES, num_subcores=SC_SUBCORES,
    )
    f = pl.kernel(
        _sc_gather_body,
        out_type=jax.ShapeDtypeStruct((EDIM, N_EDGES), jnp.float32),
        mesh=mesh,
        scratch_types=(
            [pltpu.VMEM((GCHUNK,), jnp.int32),
             pltpu.VMEM((GCHUNK,), jnp.int32),
             pltpu.VMEM((GCHUNK, EDIM), jnp.float32),
             pltpu.VMEM((GCHUNK, EDIM), jnp.float32),
             pltpu.VMEM((EDIM, GCHUNK), jnp.float32)] * 2
            + [pltpu.SemaphoreType.DMA] * 6
        ),
        compiler_params=pltpu.CompilerParams(use_tc_tiling_on_sc=False,
                                             needs_layout_passes=False),
    )
    return f(p_send, p_recv, sender, receiver)


def _epilogue_body(gt_ref, est_ref, el_ref, w1ct_ref, wlt_ref, b1t_ref,
                   w2t_ref, b2t_ref, out_ref):
    el_row = el_ref[pl.ds(pl.program_id(0), 1), :]
    z = (gt_ref[...]
         + jnp.dot(w1ct_ref[...], est_ref[...], preferred_element_type=jnp.float32)
         + wlt_ref[...] * el_row
         + b1t_ref[...])
    h = z * jax.nn.sigmoid(z)
    out_ref[...] = jnp.dot(w2t_ref[...], h,
                           preferred_element_type=jnp.float32) + b2t_ref[...]


def _epilogue(gt, es_t, el2, w1c_t, wl_t, b1_t, w2_t, b2_t):
    grid = N_EDGES // EDGE_BLK
    cblk = lambda i: (0, i)
    zblk = lambda i: (0, 0)
    return pl.pallas_call(
        _epilogue_body,
        grid=(grid,),
        in_specs=[
            pl.BlockSpec((EDIM, EDGE_BLK), cblk),
            pl.BlockSpec((EDIM, EDGE_BLK), cblk),
            pl.BlockSpec((N_EDGES // EDGE_BLK, EDGE_BLK), lambda i: (0, 0)),
            pl.BlockSpec((EDIM, EDIM), zblk),
            pl.BlockSpec((EDIM, 1), zblk),
            pl.BlockSpec((EDIM, 1), zblk),
            pl.BlockSpec((EDIM, EDIM), zblk),
            pl.BlockSpec((EDIM, 1), zblk),
        ],
        out_specs=pl.BlockSpec((EDIM, EDGE_BLK), cblk),
        out_shape=jax.ShapeDtypeStruct((EDIM, N_EDGES), jnp.float32),
    )(gt, es_t, el2, w1c_t, wl_t, b1_t, w2_t, b2_t)


@jax.jit
def kernel(scalars, edge_index, edge_len, edge_state, W1, b1, W2, b2):
    sender = edge_index[0].astype(jnp.int32)
    receiver = edge_index[1].astype(jnp.int32)
    w1a = W1[:NODE_DIM]
    w1b = W1[NODE_DIM:2 * NODE_DIM]
    w1c_t = W1[2 * NODE_DIM:2 * NODE_DIM + EDIM].T
    wl_t = W1[2 * NODE_DIM + EDIM:].T
    b1_t = b1.reshape(EDIM, 1)
    b2_t = b2.reshape(EDIM, 1)

    p_send, p_recv = _project_nodes(scalars, w1a, w1b)
    gt = _sc_gather(p_send, p_recv, sender, receiver)
    es_t = edge_state.T
    el2 = edge_len.reshape(N_EDGES // EDGE_BLK, EDGE_BLK)
    out_t = _epilogue(gt, es_t, el2, w1c_t, wl_t, b1_t, W2.T, b2_t)
    return out_t.T
